# Initial kernel scaffold; baseline (speedup 1.0000x reference)
import jax
import jax.numpy as jnp
from jax.experimental import pallas as pl


def kernel(feat_idx, w_table, r_table):
    raise NotImplementedError("write your pallas kernel here")


# speedup vs baseline: 297.7036x; 297.7036x over previous
import functools

import jax
import jax.numpy as jnp
from jax import lax
from jax.experimental import pallas as pl
from jax.experimental.pallas import tpu as pltpu
from jax.experimental.pallas import tpu_sc as plsc

W_ADJ = 0.4
R_ADJ = 10.0

NC = 2
NS = 16
LANES = 16
NW = NC * NS

VP = 100352
LP = 209
B = 16384
ROWS_PER_W = B // NW
GROUPS_PER_CHUNK = 4
CHUNK_ROWS = GROUPS_PER_CHUNK * LANES
CHUNKS = ROWS_PER_W // CHUNK_ROWS


def _prep_body(w_ref, r0_ref, r1_ref, s_ref):
    s_ref[...] = (w_ref[...] + W_ADJ) * (r1_ref[...] - r0_ref[...]) / R_ADJ


def _build_s_table(w, r0, r1):
    shaped = jax.ShapeDtypeStruct((VP // 128, 128), jnp.float32)
    f = pl.pallas_call(_prep_body, out_shape=shaped)
    return f(
        w.reshape(VP // 128, 128),
        r0.reshape(VP // 128, 128),
        r1.reshape(VP // 128, 128),
    ).reshape(VP)


def _sc_body(s_hbm, idx_hbm, out_hbm, table_v, idx_v, out_v):
    c = lax.axis_index("c")
    s = lax.axis_index("s")
    wid = s * NC + c

    pltpu.sync_copy(s_hbm, table_v)

    iota = lax.iota(jnp.int32, LANES)

    for chunk in range(CHUNKS):
        row0 = wid * ROWS_PER_W + chunk * CHUNK_ROWS
        pltpu.sync_copy(idx_hbm.at[pl.ds(row0 * LP, CHUNK_ROWS * LP)], idx_v)
        for g in range(GROUPS_PER_CHUNK):
            lane_off = iota * LP + (g * LANES * LP)

            def body(j, acc, lane_off=lane_off):
                iv = plsc.load_gather(idx_v, [lane_off + j])
                vals = plsc.load_gather(table_v, [iv])
                return acc + vals

            d = lax.fori_loop(0, LP, body, jnp.zeros((LANES,), jnp.float32))

            e = jnp.exp(-jnp.abs(d))
            inv = 1.0 / (1.0 + e)
            phi = inv
            plo = e * inv
            pos = d >= 0.0
            out0 = jnp.where(pos, plo, phi)
            out1 = jnp.where(pos, phi, plo)

            rw = chunk * CHUNK_ROWS + g * LANES
            offs = (iota + rw) * 2
            plsc.store_scatter(out_v, [offs], out0)
            plsc.store_scatter(out_v, [offs + 1], out1)

    pltpu.sync_copy(out_v, out_hbm.at[pl.ds(wid * ROWS_PER_W * 2, ROWS_PER_W * 2)])


@jax.jit
def kernel(feat_idx, w_table, r_table):
    nb, nl = feat_idx.shape
    v1 = w_table.shape[0]
    pad_v = VP - v1

    w = jnp.pad(w_table[:, 0], (0, pad_v))
    r0 = jnp.pad(r_table[:, 0], (0, pad_v))
    r1 = jnp.pad(r_table[:, 1], (0, pad_v))
    s_table = _build_s_table(w, r0, r1)

    idx = jnp.pad(feat_idx.astype(jnp.int32), ((0, 0), (0, LP - nl)))
    idx_flat = idx.reshape(-1)

    mesh = plsc.VectorSubcoreMesh(core_axis_name="c", subcore_axis_name="s")
    sc = pl.kernel(
        _sc_body,
        out_type=jax.ShapeDtypeStruct((nb * 2,), jnp.float32),
        mesh=mesh,
        scratch_types=[
            pltpu.VMEM((VP,), jnp.float32),
            pltpu.VMEM((CHUNK_ROWS * LP,), jnp.int32),
            pltpu.VMEM((ROWS_PER_W * 2,), jnp.float32),
        ],
        compiler_params=pltpu.CompilerParams(needs_layout_passes=False),
    )
    out_flat = sc(s_table, idx_flat)
    return out_flat.reshape(nb, 2)


# speedup vs baseline: 376.0165x; 1.2631x over previous
import functools

import jax
import jax.numpy as jnp
from jax import lax
from jax.experimental import pallas as pl
from jax.experimental.pallas import tpu as pltpu
from jax.experimental.pallas import tpu_sc as plsc

W_ADJ = 0.4
R_ADJ = 10.0

NC = 2
NS = 16
LANES = 16
NW = NC * NS

VP = 100352
LP = 209
B = 16384
ROWS_PER_W = B // NW
GROUPS_PER_CHUNK = 4
CHUNK_ROWS = GROUPS_PER_CHUNK * LANES
CHUNKS = ROWS_PER_W // CHUNK_ROWS


def _prep_body(w_ref, r0_ref, r1_ref, s_ref):
    s_ref[...] = (w_ref[...] + W_ADJ) * (r1_ref[...] - r0_ref[...]) / R_ADJ


def _build_s_table(w, r0, r1):
    shaped = jax.ShapeDtypeStruct((VP // 128, 128), jnp.float32)
    f = pl.pallas_call(_prep_body, out_shape=shaped)
    return f(
        w.reshape(VP // 128, 128),
        r0.reshape(VP // 128, 128),
        r1.reshape(VP // 128, 128),
    ).reshape(VP)


UNROLL = 11


def _sc_body(s_hbm, idx_hbm, out_hbm, table_v, idx_v0, idx_v1, out_v,
             sem_t, sem0, sem1):
    c = lax.axis_index("c")
    s = lax.axis_index("s")
    wid = s * NC + c
    row_base = wid * ROWS_PER_W

    bufs = (idx_v0, idx_v1)

    pltpu.sync_copy(s_hbm, table_v)

    iota = lax.iota(jnp.int32, LANES)

    for chunk in range(CHUNKS):
        buf = bufs[chunk % 2]
        flat0 = (row_base + chunk * CHUNK_ROWS) * LP
        pltpu.sync_copy(idx_hbm.at[pl.ds(flat0, CHUNK_ROWS * LP)], buf)
        for g in range(GROUPS_PER_CHUNK):
            lane_off = iota * LP + (g * LANES * LP)

            def body(j, acc, buf=buf, lane_off=lane_off):
                vs = []
                for k in range(UNROLL):
                    iv = plsc.load_gather(buf, [lane_off + (j + k)])
                    vs.append(plsc.load_gather(table_v, [iv]))
                while len(vs) > 1:
                    rest = [vs[-1]] if len(vs) % 2 else []
                    vs = [a + b for a, b in zip(vs[::2], vs[1::2])] + rest
                return acc + vs[0]

            d = plsc.parallel_loop(
                0, LP, UNROLL, carry=jnp.zeros((LANES,), jnp.float32))(body)

            e = jnp.exp(-jnp.abs(d))
            inv = 1.0 / (1.0 + e)
            phi = inv
            plo = e * inv
            pos = d >= 0.0
            out0 = jnp.where(pos, plo, phi)
            out1 = jnp.where(pos, phi, plo)

            rw = chunk * CHUNK_ROWS + g * LANES
            offs = (iota + rw) * 2
            plsc.store_scatter(out_v, [offs], out0)
            plsc.store_scatter(out_v, [offs + 1], out1)

    pltpu.sync_copy(out_v, out_hbm.at[pl.ds(wid * ROWS_PER_W * 2, ROWS_PER_W * 2)])


@jax.jit
def kernel(feat_idx, w_table, r_table):
    nb, nl = feat_idx.shape
    v1 = w_table.shape[0]
    pad_v = VP - v1

    w = jnp.pad(w_table[:, 0], (0, pad_v))
    r0 = jnp.pad(r_table[:, 0], (0, pad_v))
    r1 = jnp.pad(r_table[:, 1], (0, pad_v))
    s_table = _build_s_table(w, r0, r1)

    idx = jnp.pad(feat_idx.astype(jnp.int32), ((0, 0), (0, LP - nl)))
    idx_flat = idx.reshape(-1)

    mesh = plsc.VectorSubcoreMesh(core_axis_name="c", subcore_axis_name="s")
    sc = pl.kernel(
        _sc_body,
        out_type=jax.ShapeDtypeStruct((nb * 2,), jnp.float32),
        mesh=mesh,
        scratch_types=[
            pltpu.VMEM((VP,), jnp.float32),
            pltpu.VMEM((CHUNK_ROWS * LP,), jnp.int32),
            pltpu.VMEM((CHUNK_ROWS * LP,), jnp.int32),
            pltpu.VMEM((ROWS_PER_W * 2,), jnp.float32),
            pltpu.SemaphoreType.DMA,
            pltpu.SemaphoreType.DMA,
            pltpu.SemaphoreType.DMA,
        ],
        compiler_params=pltpu.CompilerParams(needs_layout_passes=False),
    )
    out_flat = sc(s_table, idx_flat)
    return out_flat.reshape(nb, 2)


# speedup vs baseline: 400.0369x; 1.0639x over previous
import functools

import jax
import jax.numpy as jnp
from jax import lax
from jax.experimental import pallas as pl
from jax.experimental.pallas import tpu as pltpu
from jax.experimental.pallas import tpu_sc as plsc

W_ADJ = 0.4
R_ADJ = 10.0

NC = 2
NS = 16
LANES = 16
NW = NC * NS

VP = 100352
LP = 200
PHASE = 13
B = 16384
ROWS_PER_W = B // NW
GROUPS_PER_CHUNK = 4
CHUNK_ROWS = GROUPS_PER_CHUNK * LANES
CHUNKS = ROWS_PER_W // CHUNK_ROWS


def _prep_body(w_ref, r0_ref, r1_ref, s_ref):
    s_ref[...] = (w_ref[...] + W_ADJ) * (r1_ref[...] - r0_ref[...]) / R_ADJ


def _build_s_table(w, r0, r1):
    shaped = jax.ShapeDtypeStruct((VP // 128, 128), jnp.float32)
    f = pl.pallas_call(_prep_body, out_shape=shaped)
    return f(
        w.reshape(VP // 128, 128),
        r0.reshape(VP // 128, 128),
        r1.reshape(VP // 128, 128),
    ).reshape(VP)


UNROLL = 10


def _sc_body(s_hbm, idx_hbm, out_hbm, table_v, idx_v0, idx_v1, out_v,
             sem_t, sem0, sem1):
    c = lax.axis_index("c")
    s = lax.axis_index("s")
    wid = s * NC + c
    row_base = wid * ROWS_PER_W

    bufs = (idx_v0, idx_v1)

    pltpu.sync_copy(s_hbm, table_v)

    iota = lax.iota(jnp.int32, LANES)

    for chunk in range(CHUNKS):
        buf = bufs[chunk % 2]
        flat0 = (row_base + chunk * CHUNK_ROWS) * LP
        pltpu.sync_copy(idx_hbm.at[pl.ds(flat0, CHUNK_ROWS * LP)], buf)
        thresh = LP - PHASE * iota
        for g in range(GROUPS_PER_CHUNK):
            lane_off = iota * (LP + PHASE) + (g * LANES * LP)

            def body(j, acc, buf=buf, lane_off=lane_off):
                vs = []
                for k in range(UNROLL):
                    jj = j + k
                    pos = lane_off + jj
                    offs = jnp.where(thresh <= jj, pos - LP, pos)
                    iv = plsc.load_gather(buf, [offs])
                    vs.append(plsc.load_gather(table_v, [iv]))
                while len(vs) > 1:
                    rest = [vs[-1]] if len(vs) % 2 else []
                    vs = [a + b for a, b in zip(vs[::2], vs[1::2])] + rest
                return acc + vs[0]

            d = plsc.parallel_loop(
                0, LP, UNROLL, carry=jnp.zeros((LANES,), jnp.float32))(body)

            e = jnp.exp(-jnp.abs(d))
            inv = 1.0 / (1.0 + e)
            phi = inv
            plo = e * inv
            pos = d >= 0.0
            out0 = jnp.where(pos, plo, phi)
            out1 = jnp.where(pos, phi, plo)

            rw = chunk * CHUNK_ROWS + g * LANES
            offs = (iota + rw) * 2
            plsc.store_scatter(out_v, [offs], out0)
            plsc.store_scatter(out_v, [offs + 1], out1)

    pltpu.sync_copy(out_v, out_hbm.at[pl.ds(wid * ROWS_PER_W * 2, ROWS_PER_W * 2)])


@jax.jit
def kernel(feat_idx, w_table, r_table):
    nb, nl = feat_idx.shape
    v1 = w_table.shape[0]
    pad_v = VP - v1

    w = jnp.pad(w_table[:, 0], (0, pad_v))
    r0 = jnp.pad(r_table[:, 0], (0, pad_v))
    r1 = jnp.pad(r_table[:, 1], (0, pad_v))
    s_table = _build_s_table(w, r0, r1)

    idx_flat = feat_idx.astype(jnp.int32).reshape(-1)

    mesh = plsc.VectorSubcoreMesh(core_axis_name="c", subcore_axis_name="s")
    sc = pl.kernel(
        _sc_body,
        out_type=jax.ShapeDtypeStruct((nb * 2,), jnp.float32),
        mesh=mesh,
        scratch_types=[
            pltpu.VMEM((VP,), jnp.float32),
            pltpu.VMEM((CHUNK_ROWS * LP,), jnp.int32),
            pltpu.VMEM((CHUNK_ROWS * LP,), jnp.int32),
            pltpu.VMEM((ROWS_PER_W * 2,), jnp.float32),
            pltpu.SemaphoreType.DMA,
            pltpu.SemaphoreType.DMA,
            pltpu.SemaphoreType.DMA,
        ],
        compiler_params=pltpu.CompilerParams(needs_layout_passes=False),
    )
    out_flat = sc(s_table, idx_flat)
    return out_flat.reshape(nb, 2)


# speedup vs baseline: 402.9533x; 1.0073x over previous
import jax
import jax.numpy as jnp
from jax import lax
from jax.experimental import pallas as pl
from jax.experimental.pallas import tpu as pltpu
from jax.experimental.pallas import tpu_sc as plsc

W_ADJ = 0.4
R_ADJ = 10.0

NC = 2
NS = 16
LANES = 16
NW = NC * NS

VP = 100352
LP = 200
PHASE = 13
B = 16384
ROWS_PER_W = B // NW
GROUPS_PER_CHUNK = 2
CHUNK_ROWS = GROUPS_PER_CHUNK * LANES
CHUNKS = ROWS_PER_W // CHUNK_ROWS
UNROLL = 10


def _prep_body(w_ref, r0_ref, r1_ref, s_ref):
    s_ref[...] = (w_ref[...] + W_ADJ) * (r1_ref[...] - r0_ref[...]) / R_ADJ


def _build_s_table(w, r0, r1):
    shaped = jax.ShapeDtypeStruct((VP // 128, 128), jnp.float32)
    f = pl.pallas_call(_prep_body, out_shape=shaped)
    return f(
        w.reshape(VP // 128, 128),
        r0.reshape(VP // 128, 128),
        r1.reshape(VP // 128, 128),
    ).reshape(VP)


def _sc_body(s_hbm, idx_hbm, out_hbm, table_v, idx_v0, idx_v1, out_v):
    c = lax.axis_index("c")
    s = lax.axis_index("s")
    wid = s * NC + c
    row_base = wid * ROWS_PER_W

    bufs = (idx_v0, idx_v1)

    pltpu.sync_copy(s_hbm, table_v)

    iota = lax.iota(jnp.int32, LANES)
    thresh = LP - PHASE * iota

    for chunk in range(CHUNKS):
        buf = bufs[chunk % 2]
        row0 = row_base + chunk * CHUNK_ROWS
        pltpu.sync_copy(idx_hbm.at[pl.ds(row0 * LP, CHUNK_ROWS * LP)], buf)
        for g in range(GROUPS_PER_CHUNK):
            lane_off = iota * (LP + PHASE) + (g * LANES * LP)

            def body(j, acc, buf=buf, lane_off=lane_off):
                vs = []
                for k in range(UNROLL):
                    jj = j + k
                    pos = lane_off + jj
                    offs = jnp.where(thresh <= jj, pos - LP, pos)
                    iv = plsc.load_gather(buf, [offs])
                    vs.append(plsc.load_gather(table_v, [iv]))
                while len(vs) > 1:
                    rest = [vs[-1]] if len(vs) % 2 else []
                    vs = [a + b for a, b in zip(vs[::2], vs[1::2])] + rest
                return acc + vs[0]

            d = plsc.parallel_loop(
                0, LP, UNROLL, carry=jnp.zeros((LANES,), jnp.float32))(body)

            e = jnp.exp(-jnp.abs(d))
            inv = 1.0 / (1.0 + e)
            phi = inv
            plo = e * inv
            pos_m = d >= 0.0
            out0 = jnp.where(pos_m, plo, phi)
            out1 = jnp.where(pos_m, phi, plo)

            rows_l = iota + g * LANES
            plsc.store_scatter(out_v, [rows_l, jnp.zeros((LANES,), jnp.int32)],
                               out0)
            plsc.store_scatter(out_v, [rows_l, jnp.ones((LANES,), jnp.int32)],
                               out1)

        pltpu.sync_copy(out_v, out_hbm.at[pl.ds(row0, CHUNK_ROWS)])


@jax.jit
def kernel(feat_idx, w_table, r_table):
    nb, nl = feat_idx.shape
    v1 = w_table.shape[0]
    pad_v = VP - v1

    w = jnp.pad(w_table[:, 0], (0, pad_v))
    r0 = jnp.pad(r_table[:, 0], (0, pad_v))
    r1 = jnp.pad(r_table[:, 1], (0, pad_v))
    s_table = _build_s_table(w, r0, r1)

    idx_flat = feat_idx.astype(jnp.int32).reshape(-1)

    mesh = plsc.VectorSubcoreMesh(core_axis_name="c", subcore_axis_name="s")
    sc = pl.kernel(
        _sc_body,
        out_type=jax.ShapeDtypeStruct((nb, 2), jnp.float32),
        mesh=mesh,
        scratch_types=[
            pltpu.VMEM((VP,), jnp.float32),
            pltpu.VMEM((CHUNK_ROWS * LP,), jnp.int32),
            pltpu.VMEM((CHUNK_ROWS * LP,), jnp.int32),
            pltpu.VMEM((CHUNK_ROWS, 2), jnp.float32),
        ],
        compiler_params=pltpu.CompilerParams(needs_layout_passes=False),
    )
    return sc(s_table, idx_flat)


# speedup vs baseline: 426.8488x; 1.0593x over previous
import jax
import jax.numpy as jnp
from jax import lax
from jax.experimental import pallas as pl
from jax.experimental.pallas import tpu as pltpu
from jax.experimental.pallas import tpu_sc as plsc

W_ADJ = 0.4
R_ADJ = 10.0

NC = 2
NS = 16
LANES = 16
NW = NC * NS

VP = 100352
LP = 200
PHASE = 13
B = 16384
ROWS_PER_W = B // NW
GROUPS_PER_CHUNK = 4
CHUNK_ROWS = GROUPS_PER_CHUNK * LANES
CHUNKS = ROWS_PER_W // CHUNK_ROWS
UNROLL = 10


def _prep_body(w_ref, r0_ref, r1_ref, s_ref):
    s_ref[...] = (w_ref[...] + W_ADJ) * (r1_ref[...] - r0_ref[...]) / R_ADJ


def _build_s_table(w, r0, r1):
    shaped = jax.ShapeDtypeStruct((VP // 128, 128), jnp.float32)
    f = pl.pallas_call(_prep_body, out_shape=shaped)
    return f(
        w.reshape(VP // 128, 128),
        r0.reshape(VP // 128, 128),
        r1.reshape(VP // 128, 128),
    ).reshape(VP)


def _sc_body(s_hbm, idx_hbm, out_hbm, table_v, idx_v, out_v):
    c = lax.axis_index("c")
    s = lax.axis_index("s")
    wid = s * NC + c
    row_base = wid * ROWS_PER_W

    pltpu.sync_copy(s_hbm, table_v)

    iota = lax.iota(jnp.int32, LANES)
    thresh = LP - PHASE * iota

    for chunk in range(CHUNKS):
        buf = idx_v
        row0 = row_base + chunk * CHUNK_ROWS
        pltpu.sync_copy(idx_hbm.at[pl.ds(row0 * LP, CHUNK_ROWS * LP)], buf)
        for g in range(GROUPS_PER_CHUNK):
            lane_off = iota * (LP + PHASE) + (g * LANES * LP)

            def body(j, acc, buf=buf, lane_off=lane_off):
                vs = []
                for k in range(UNROLL):
                    jj = j + k
                    pos = lane_off + jj
                    offs = jnp.where(thresh <= jj, pos - LP, pos)
                    iv = plsc.load_gather(buf, [offs])
                    vs.append(plsc.load_gather(table_v, [iv]))
                while len(vs) > 1:
                    rest = [vs[-1]] if len(vs) % 2 else []
                    vs = [a + b for a, b in zip(vs[::2], vs[1::2])] + rest
                return acc + vs[0]

            d = plsc.parallel_loop(
                0, LP, UNROLL, carry=jnp.zeros((LANES,), jnp.float32))(body)

            e = jnp.exp(-jnp.abs(d))
            inv = 1.0 / (1.0 + e)
            phi = inv
            plo = e * inv
            pos_m = d >= 0.0
            out0 = jnp.where(pos_m, plo, phi)
            out1 = jnp.where(pos_m, phi, plo)

            rows_l = iota + g * LANES
            plsc.store_scatter(out_v, [rows_l, jnp.zeros((LANES,), jnp.int32)],
                               out0)
            plsc.store_scatter(out_v, [rows_l, jnp.ones((LANES,), jnp.int32)],
                               out1)

        pltpu.sync_copy(out_v, out_hbm.at[pl.ds(row0, CHUNK_ROWS)])


@jax.jit
def kernel(feat_idx, w_table, r_table):
    nb, nl = feat_idx.shape
    v1 = w_table.shape[0]
    pad_v = VP - v1

    w = jnp.pad(w_table[:, 0], (0, pad_v))
    r0 = jnp.pad(r_table[:, 0], (0, pad_v))
    r1 = jnp.pad(r_table[:, 1], (0, pad_v))
    s_table = _build_s_table(w, r0, r1)

    idx_lin = lax.optimization_barrier(
        feat_idx.astype(jnp.int32).reshape(nb * nl // 128, 128))
    idx_flat = idx_lin.reshape(-1)

    mesh = plsc.VectorSubcoreMesh(core_axis_name="c", subcore_axis_name="s")
    sc = pl.kernel(
        _sc_body,
        out_type=jax.ShapeDtypeStruct((nb, 2), jnp.float32),
        mesh=mesh,
        scratch_types=[
            pltpu.VMEM((VP,), jnp.float32),
            pltpu.VMEM((CHUNK_ROWS * LP,), jnp.int32),
            pltpu.VMEM((CHUNK_ROWS, 2), jnp.float32),
        ],
        compiler_params=pltpu.CompilerParams(needs_layout_passes=False),
    )
    return sc(s_table, idx_flat)


# speedup vs baseline: 469.0121x; 1.0988x over previous
import jax
import jax.numpy as jnp
from jax import lax
from jax.experimental import pallas as pl
from jax.experimental.pallas import tpu as pltpu
from jax.experimental.pallas import tpu_sc as plsc

W_ADJ = 0.4
R_ADJ = 10.0

NC = 2
NS = 16
LANES = 16
NW = NC * NS

VP = 100352
LP = 200
PHASE = 13
B = 16384
ROWS_PER_W = B // NW
GROUPS_PER_CHUNK = 4
CHUNK_ROWS = GROUPS_PER_CHUNK * LANES
CHUNKS = ROWS_PER_W // CHUNK_ROWS
UNROLL = 10


def _prep_body(w_ref, r0_ref, r1_ref, s_ref):
    s_ref[...] = (w_ref[...] + W_ADJ) * (r1_ref[...] - r0_ref[...]) / R_ADJ


def _build_s_table(w, r0, r1):
    shaped = jax.ShapeDtypeStruct((VP // 128, 128), jnp.float32)
    f = pl.pallas_call(_prep_body, out_shape=shaped)
    return f(
        w.reshape(VP // 128, 128),
        r0.reshape(VP // 128, 128),
        r1.reshape(VP // 128, 128),
    ).reshape(VP)


def _sc_body(s_hbm, idx_hbm, out_hbm, table_v, idx_v, out_v):
    c = lax.axis_index("c")
    s = lax.axis_index("s")
    wid = s * NC + c
    row_base = wid * ROWS_PER_W

    pltpu.sync_copy(s_hbm, table_v)

    iota = lax.iota(jnp.int32, LANES)
    thresh = LP - PHASE * iota

    phase = PHASE * iota

    for chunk in range(CHUNKS):
        buf = idx_v
        row0 = row_base + chunk * CHUNK_ROWS
        pltpu.sync_copy(idx_hbm.at[pl.ds(row0, CHUNK_ROWS)], buf)
        for g in range(GROUPS_PER_CHUNK):
            rows = iota + g * LANES

            def body(j, acc, buf=buf, rows=rows):
                vs = []
                for k in range(UNROLL):
                    jj = j + k
                    pos = phase + jj
                    col = jnp.where(thresh <= jj, pos - LP, pos)
                    iv = plsc.load_gather(buf, [rows, col])
                    vs.append(plsc.load_gather(table_v, [iv]))
                while len(vs) > 1:
                    rest = [vs[-1]] if len(vs) % 2 else []
                    vs = [a + b for a, b in zip(vs[::2], vs[1::2])] + rest
                return acc + vs[0]

            d = plsc.parallel_loop(
                0, LP, UNROLL, carry=jnp.zeros((LANES,), jnp.float32))(body)

            e = jnp.exp(-jnp.abs(d))
            inv = 1.0 / (1.0 + e)
            phi = inv
            plo = e * inv
            pos_m = d >= 0.0
            out0 = jnp.where(pos_m, plo, phi)
            out1 = jnp.where(pos_m, phi, plo)

            rows_l = iota + g * LANES
            plsc.store_scatter(out_v, [rows_l, jnp.zeros((LANES,), jnp.int32)],
                               out0)
            plsc.store_scatter(out_v, [rows_l, jnp.ones((LANES,), jnp.int32)],
                               out1)

        pltpu.sync_copy(out_v, out_hbm.at[pl.ds(row0, CHUNK_ROWS)])


@jax.jit
def kernel(feat_idx, w_table, r_table):
    nb, nl = feat_idx.shape
    v1 = w_table.shape[0]
    pad_v = VP - v1

    w = jnp.pad(w_table[:, 0], (0, pad_v))
    r0 = jnp.pad(r_table[:, 0], (0, pad_v))
    r1 = jnp.pad(r_table[:, 1], (0, pad_v))
    s_table = _build_s_table(w, r0, r1)

    idx2d = feat_idx.astype(jnp.int32)

    mesh = plsc.VectorSubcoreMesh(core_axis_name="c", subcore_axis_name="s")
    sc = pl.kernel(
        _sc_body,
        out_type=jax.ShapeDtypeStruct((nb, 2), jnp.float32),
        mesh=mesh,
        scratch_types=[
            pltpu.VMEM((VP,), jnp.float32),
            pltpu.VMEM((CHUNK_ROWS, LP), jnp.int32),
            pltpu.VMEM((CHUNK_ROWS, 2), jnp.float32),
        ],
        compiler_params=pltpu.CompilerParams(needs_layout_passes=False),
    )
    return sc(s_table, idx2d)


# speedup vs baseline: 520.8611x; 1.1105x over previous
import jax
import jax.numpy as jnp
from jax import lax
from jax.experimental import pallas as pl
from jax.experimental.pallas import tpu as pltpu
from jax.experimental.pallas import tpu_sc as plsc

W_ADJ = 0.4
R_ADJ = 10.0

NC = 2
NS = 16
LANES = 16
NW = NC * NS

VP = 100352
LP = 200
PHASE = 13
B = 16384
ROWS_PER_W = B // NW
GROUPS_PER_CHUNK = 2
CHUNK_ROWS = GROUPS_PER_CHUNK * LANES
CHUNKS = ROWS_PER_W // CHUNK_ROWS
UNROLL = 10


def _prep_body(w_ref, r0_ref, r1_ref, s_ref):
    s_ref[...] = (w_ref[...] + W_ADJ) * (r1_ref[...] - r0_ref[...]) / R_ADJ


def _build_s_table(w, r0, r1):
    shaped = jax.ShapeDtypeStruct((VP // 128, 128), jnp.float32)
    f = pl.pallas_call(_prep_body, out_shape=shaped)
    return f(
        w.reshape(VP // 128, 128),
        r0.reshape(VP // 128, 128),
        r1.reshape(VP // 128, 128),
    ).reshape(VP)


def _sc_body(s_hbm, idx_hbm, out_hbm, table_v, idx_v0, idx_v1, out_v,
             sem0, sem1):
    c = lax.axis_index("c")
    s = lax.axis_index("s")
    wid = s * NC + c
    row_base = wid * ROWS_PER_W

    pltpu.sync_copy(s_hbm, table_v)

    iota = lax.iota(jnp.int32, LANES)
    thresh = LP - PHASE * iota

    phase = PHASE * iota

    bufs = (idx_v0, idx_v1)
    sems = (sem0, sem1)

    def fire(chunk):
        row0 = row_base + chunk * CHUNK_ROWS
        return pltpu.async_copy(
            idx_hbm.at[pl.ds(row0, CHUNK_ROWS)], bufs[chunk % 2],
            sems[chunk % 2])

    desc = fire(0)
    for chunk in range(CHUNKS):
        buf = bufs[chunk % 2]
        row0 = row_base + chunk * CHUNK_ROWS
        desc.wait()
        if chunk + 1 < CHUNKS:
            desc = fire(chunk + 1)
        for g in range(GROUPS_PER_CHUNK):
            rows = iota + g * LANES

            def body(j, acc, buf=buf, rows=rows):
                vs = []
                for k in range(UNROLL):
                    jj = j + k
                    pos = phase + jj
                    col = jnp.where(thresh <= jj, pos - LP, pos)
                    iv = plsc.load_gather(buf, [rows, col])
                    vs.append(plsc.load_gather(table_v, [iv]))
                while len(vs) > 1:
                    rest = [vs[-1]] if len(vs) % 2 else []
                    vs = [a + b for a, b in zip(vs[::2], vs[1::2])] + rest
                return acc + vs[0]

            d = plsc.parallel_loop(
                0, LP, UNROLL, carry=jnp.zeros((LANES,), jnp.float32))(body)

            e = jnp.exp(-jnp.abs(d))
            inv = 1.0 / (1.0 + e)
            phi = inv
            plo = e * inv
            pos_m = d >= 0.0
            out0 = jnp.where(pos_m, plo, phi)
            out1 = jnp.where(pos_m, phi, plo)

            rows_l = iota + g * LANES
            plsc.store_scatter(out_v, [rows_l, jnp.zeros((LANES,), jnp.int32)],
                               out0)
            plsc.store_scatter(out_v, [rows_l, jnp.ones((LANES,), jnp.int32)],
                               out1)

        pltpu.sync_copy(out_v, out_hbm.at[pl.ds(row0, CHUNK_ROWS)])


@jax.jit
def kernel(feat_idx, w_table, r_table):
    nb, nl = feat_idx.shape
    v1 = w_table.shape[0]
    pad_v = VP - v1

    w = jnp.pad(w_table[:, 0], (0, pad_v))
    r0 = jnp.pad(r_table[:, 0], (0, pad_v))
    r1 = jnp.pad(r_table[:, 1], (0, pad_v))
    s_table = _build_s_table(w, r0, r1)

    idx2d = feat_idx.astype(jnp.int32)

    mesh = plsc.VectorSubcoreMesh(core_axis_name="c", subcore_axis_name="s")
    sc = pl.kernel(
        _sc_body,
        out_type=jax.ShapeDtypeStruct((nb, 2), jnp.float32),
        mesh=mesh,
        scratch_types=[
            pltpu.VMEM((VP,), jnp.float32),
            pltpu.VMEM((CHUNK_ROWS, LP), jnp.int32),
            pltpu.VMEM((CHUNK_ROWS, LP), jnp.int32),
            pltpu.VMEM((CHUNK_ROWS, 2), jnp.float32),
            pltpu.SemaphoreType.DMA,
            pltpu.SemaphoreType.DMA,
        ],
        compiler_params=pltpu.CompilerParams(needs_layout_passes=False),
    )
    return sc(s_table, idx2d)
